# Initial kernel scaffold; baseline (speedup 1.0000x reference)
import jax
import jax.numpy as jnp
from jax.experimental import pallas as pl


def kernel(node_coord_src, node_feature_src, batch_src):
    raise NotImplementedError("write your pallas kernel here")


# speedup vs baseline: 14.4026x; 14.4026x over previous
import functools

import jax
import jax.numpy as jnp
from jax.experimental import pallas as pl
from jax.experimental.pallas import tpu as pltpu

_K = 16
_TR = 256
_TC = 512
_BIG = 2**30


def _knn_body(lo_ref, cnt_ref, coords_ref, coordsT_ref, brow_ref, bcol_ref,
              out_src_ref, out_aux_ref, cand_val, cand_idx, bval, bidx):
    i = pl.program_id(0)
    dst = coords_ref[pl.ds(i * _TR, _TR), :]
    bd = brow_ref[pl.ds(i * _TR, _TR), :]
    sqd = jnp.sum(dst * dst, axis=1, keepdims=True)
    rows = i * _TR + jax.lax.broadcasted_iota(jnp.int32, (_TR, 1), 0)

    bval[...] = jnp.full((_TR, _K), jnp.inf, jnp.float32)
    bidx[...] = jnp.full((_TR, _K), _BIG, jnp.int32)
    cand_val[:, _TC:] = jnp.full((_TR, 128), jnp.inf, jnp.float32)
    cand_idx[:, _TC:] = jnp.full((_TR, 128), _BIG, jnp.int32)

    def col_body(j, carry):
        c0 = (lo_ref[i] + j) * _TC
        src = coordsT_ref[:, pl.ds(c0, _TC)]
        bs = bcol_ref[:, pl.ds(c0, _TC)]
        m2 = jnp.dot(dst, src, preferred_element_type=jnp.float32)
        sqs = jnp.sum(src * src, axis=0, keepdims=True)
        d2 = (sqd + sqs) - 2.0 * m2
        d2 = jnp.where(bd == bs, d2, jnp.inf)
        cand_val[:, 0:_TC] = d2
        cand_idx[:, 0:_TC] = c0 + jax.lax.broadcasted_iota(
            jnp.int32, (_TR, _TC), 1)
        cand_val[:, _TC:_TC + _K] = bval[...]
        cand_idx[:, _TC:_TC + _K] = bidx[...]
        for r in range(_K):
            cv = cand_val[...]
            ci = cand_idx[...]
            m = jnp.min(cv, axis=1, keepdims=True)
            ismin = cv == m
            si = jnp.min(jnp.where(ismin, ci, _BIG), axis=1, keepdims=True)
            bval[:, r:r + 1] = m
            bidx[:, r:r + 1] = si
            cand_val[...] = jnp.where(ismin & (ci == si), jnp.inf, cv)
        return carry

    jax.lax.fori_loop(0, cnt_ref[i], col_body, 0)

    bi = bidx[...]
    selfm = bi == rows
    lane = jax.lax.broadcasted_iota(jnp.int32, (_TR, _K), 1)
    p = jnp.min(jnp.where(selfm, lane, _K), axis=1, keepdims=True)
    kept = jnp.where(lane[:, :_K - 1] < p, bi[:, :_K - 1], bi[:, 1:_K])
    out_src_ref[:, 0:_K - 1] = kept
    out_src_ref[:, _K - 1:_K] = jnp.zeros((_TR, 1), jnp.int32)
    deg = _K - jnp.sum(selfm.astype(jnp.int32), axis=1, keepdims=True)
    out_aux_ref[:, 0:_K - 1] = jnp.broadcast_to(rows, (_TR, _K - 1))
    out_aux_ref[:, _K - 1:_K] = deg


@jax.jit
def _knn_pallas(node_coord_src, batch_src):
    n = node_coord_src.shape[0]
    npad = ((n + _TC - 1) // _TC) * _TC
    nt = npad // _TR
    nb = 8

    coords8 = jnp.zeros((npad, 8), jnp.float32)
    coords8 = coords8.at[:n, :3].set(node_coord_src)
    coordsT = coords8.T
    brow = jnp.full((npad, 1), -1, jnp.int32).at[:n, 0].set(batch_src)
    bcol = jnp.full((1, npad), -2, jnp.int32).at[0, :n].set(batch_src)

    starts = jnp.searchsorted(batch_src, jnp.arange(nb, dtype=jnp.int32),
                              side="left").astype(jnp.int32)
    ends = jnp.searchsorted(batch_src, jnp.arange(nb, dtype=jnp.int32),
                            side="right").astype(jnp.int32)
    first = jnp.minimum(jnp.arange(nt, dtype=jnp.int32) * _TR, n - 1)
    last = jnp.minimum(first + _TR - 1, n - 1)
    lo_t = starts[batch_src[first]] // _TC
    hi_t = (ends[batch_src[last]] - 1) // _TC
    cnt_t = hi_t - lo_t + 1

    grid_spec = pltpu.PrefetchScalarGridSpec(
        num_scalar_prefetch=2,
        grid=(nt,),
        in_specs=[
            pl.BlockSpec((npad, 8), lambda i, lo, cnt: (0, 0)),
            pl.BlockSpec((8, npad), lambda i, lo, cnt: (0, 0)),
            pl.BlockSpec((npad, 1), lambda i, lo, cnt: (0, 0)),
            pl.BlockSpec((1, npad), lambda i, lo, cnt: (0, 0)),
        ],
        out_specs=[
            pl.BlockSpec((_TR, _K), lambda i, lo, cnt: (i, 0)),
            pl.BlockSpec((_TR, _K), lambda i, lo, cnt: (i, 0)),
        ],
        scratch_shapes=[
            pltpu.VMEM((_TR, _TC + 128), jnp.float32),
            pltpu.VMEM((_TR, _TC + 128), jnp.int32),
            pltpu.VMEM((_TR, _K), jnp.float32),
            pltpu.VMEM((_TR, _K), jnp.int32),
        ],
    )
    out_src, out_aux = pl.pallas_call(
        _knn_body,
        grid_spec=grid_spec,
        out_shape=[
            jax.ShapeDtypeStruct((nt * _TR, _K), jnp.int32),
            jax.ShapeDtypeStruct((nt * _TR, _K), jnp.int32),
        ],
    )(lo_t, cnt_t, coords8, coordsT, brow, bcol)
    return out_src, out_aux


def kernel(node_coord_src, node_feature_src, batch_src):
    n = node_coord_src.shape[0]
    out_src, out_aux = _knn_pallas(node_coord_src, batch_src)
    edge_src = out_src[:n, :_K - 1].reshape(-1)
    edge_dst = out_aux[:n, :_K - 1].reshape(-1)
    degree = out_aux[:n, _K - 1]
    return (node_feature_src, node_coord_src, edge_src, edge_dst, degree,
            batch_src)


# speedup vs baseline: 25.7093x; 1.7850x over previous
import jax
import jax.numpy as jnp
from jax.experimental import pallas as pl
from jax.experimental.pallas import tpu as pltpu

_K = 16
_TR = 256
_TC = 512
_W = _K + _TC
_BIG = 2**30
_INF = jnp.inf


def _knn_body(lo_ref, cnt_ref, coords_ref, coordsT_ref, brow_ref, bcol_ref,
              out_src_ref, out_aux_ref, cand_val, cand_idx, bval, bidx):
    i = pl.program_id(0)
    dstT = coordsT_ref[:, pl.ds(i * _TR, _TR)]
    bd = bcol_ref[:, pl.ds(i * _TR, _TR)]
    sqd = jnp.sum(dstT * dstT, axis=0, keepdims=True)
    nodes = i * _TR + jax.lax.broadcasted_iota(jnp.int32, (1, _TR), 1)

    bval[...] = jnp.full((_K, _TR), _INF, jnp.float32)
    bidx[...] = jnp.full((_K, _TR), _BIG, jnp.int32)

    sub16 = jax.lax.broadcasted_iota(jnp.int32, (_K, _TR), 0)
    subW = jax.lax.broadcasted_iota(jnp.int32, (_W, _TR), 0)

    def col_body(j, carry):
        c0 = (lo_ref[i] + j) * _TC
        src = coords_ref[pl.ds(c0, _TC), :]
        bs = brow_ref[pl.ds(c0, _TC), :]
        m2 = jnp.dot(src, dstT, preferred_element_type=jnp.float32)
        sqs = jnp.sum(src * src, axis=1, keepdims=True)
        d2 = (sqd + sqs) - 2.0 * m2
        d2 = jnp.where(bs == bd, d2, _INF)
        cand_val[0:_K, :] = bval[...]
        cand_idx[0:_K, :] = bidx[...]
        cand_val[_K:_W, :] = d2
        ci16 = cand_idx[0:_K, :]
        for r in range(_K):
            cv = cand_val[...]
            m = jnp.min(cv, axis=0, keepdims=True)
            pos = jnp.argmin(cv, axis=0)[None, :]
            gi = jnp.min(jnp.where(sub16 == pos, ci16, _BIG),
                         axis=0, keepdims=True)
            bval[r:r + 1, :] = m
            bidx[r:r + 1, :] = jnp.where(pos < _K, gi, c0 + pos - _K)
            cand_val[...] = jnp.where(subW == pos, _INF, cv)
        return carry

    jax.lax.fori_loop(0, cnt_ref[i], col_body, 0)

    bi = bidx[...]
    selfm = bi == nodes
    p = jnp.min(jnp.where(selfm, sub16, _K), axis=0, keepdims=True)
    kept = jnp.where(sub16[:_K - 1, :] < p, bi[:_K - 1, :], bi[1:_K, :])
    out_src_ref[0:_K - 1, :] = kept
    out_src_ref[_K - 1:_K, :] = jnp.zeros((1, _TR), jnp.int32)
    deg = _K - jnp.sum(selfm.astype(jnp.int32), axis=0, keepdims=True)
    out_aux_ref[0:_K - 1, :] = jnp.broadcast_to(nodes, (_K - 1, _TR))
    out_aux_ref[_K - 1:_K, :] = deg


@jax.jit
def _knn_pallas(node_coord_src, batch_src):
    n = node_coord_src.shape[0]
    npad = ((n + _TC - 1) // _TC) * _TC
    nt = npad // _TR
    nb = 8

    coords8 = jnp.zeros((npad, 8), jnp.float32)
    coords8 = coords8.at[:n, :3].set(node_coord_src)
    coordsT = coords8.T
    brow = jnp.full((npad, 1), -1, jnp.int32).at[:n, 0].set(batch_src)
    bcol = jnp.full((1, npad), -2, jnp.int32).at[0, :n].set(batch_src)

    starts = jnp.searchsorted(batch_src, jnp.arange(nb, dtype=jnp.int32),
                              side="left").astype(jnp.int32)
    ends = jnp.searchsorted(batch_src, jnp.arange(nb, dtype=jnp.int32),
                            side="right").astype(jnp.int32)
    first = jnp.minimum(jnp.arange(nt, dtype=jnp.int32) * _TR, n - 1)
    last = jnp.minimum(first + _TR - 1, n - 1)
    lo_t = starts[batch_src[first]] // _TC
    hi_t = (ends[batch_src[last]] - 1) // _TC
    cnt_t = hi_t - lo_t + 1

    grid_spec = pltpu.PrefetchScalarGridSpec(
        num_scalar_prefetch=2,
        grid=(nt,),
        in_specs=[
            pl.BlockSpec((npad, 8), lambda i, lo, cnt: (0, 0)),
            pl.BlockSpec((8, npad), lambda i, lo, cnt: (0, 0)),
            pl.BlockSpec((npad, 1), lambda i, lo, cnt: (0, 0)),
            pl.BlockSpec((1, npad), lambda i, lo, cnt: (0, 0)),
        ],
        out_specs=[
            pl.BlockSpec((_K, _TR), lambda i, lo, cnt: (0, i)),
            pl.BlockSpec((_K, _TR), lambda i, lo, cnt: (0, i)),
        ],
        scratch_shapes=[
            pltpu.VMEM((_W, _TR), jnp.float32),
            pltpu.VMEM((_W, _TR), jnp.int32),
            pltpu.VMEM((_K, _TR), jnp.float32),
            pltpu.VMEM((_K, _TR), jnp.int32),
        ],
    )
    out_src, out_aux = pl.pallas_call(
        _knn_body,
        grid_spec=grid_spec,
        out_shape=[
            jax.ShapeDtypeStruct((_K, nt * _TR), jnp.int32),
            jax.ShapeDtypeStruct((_K, nt * _TR), jnp.int32),
        ],
    )(lo_t, cnt_t, coords8, coordsT, brow, bcol)
    return out_src, out_aux


def kernel(node_coord_src, node_feature_src, batch_src):
    n = node_coord_src.shape[0]
    out_src, out_aux = _knn_pallas(node_coord_src, batch_src)
    edge_src = out_src[:_K - 1, :n].T.reshape(-1)
    edge_dst = out_aux[:_K - 1, :n].T.reshape(-1)
    degree = out_aux[_K - 1, :n]
    return (node_feature_src, node_coord_src, edge_src, edge_dst, degree,
            batch_src)


# speedup vs baseline: 26.4921x; 1.0304x over previous
import jax
import jax.numpy as jnp
from jax.experimental import pallas as pl
from jax.experimental.pallas import tpu as pltpu

_K = 16
_TR = 256
_TC = 512
_W = _K + _TC
_BIG = 2**30
_INF = jnp.inf


def _knn_body(lo_ref, cnt_ref, coords_ref, coordsT_ref, brow_ref, bcol_ref,
              out_src_ref, out_aux_ref):
    i = pl.program_id(0)
    dstT = coordsT_ref[:, pl.ds(i * _TR, _TR)]
    dstT2 = -2.0 * dstT
    bd = bcol_ref[:, pl.ds(i * _TR, _TR)]
    sqd = jnp.sum(dstT * dstT, axis=0, keepdims=True)
    nodes = i * _TR + jax.lax.broadcasted_iota(jnp.int32, (1, _TR), 1)

    sub16 = jax.lax.broadcasted_iota(jnp.int32, (_K, _TR), 0)
    subW = jax.lax.broadcasted_iota(jnp.int32, (_W, _TR), 0)

    def col_body(j, carry):
        bval, bidx = carry
        c0 = (lo_ref[i] + j) * _TC
        src = coords_ref[pl.ds(c0, _TC), :]
        bs = brow_ref[pl.ds(c0, _TC), :]
        m2 = jnp.dot(src, dstT2, preferred_element_type=jnp.float32)
        sqs = jnp.sum(src * src, axis=1, keepdims=True)
        d2 = (sqd + sqs) + m2
        d2 = jnp.where(bs == bd, d2, _INF)
        cv = jnp.concatenate([bval, d2], axis=0)
        vals, idxs = [], []
        for r in range(_K):
            m = jnp.min(cv, axis=0, keepdims=True)
            pos = jnp.argmin(cv, axis=0)[None, :]
            gi = jnp.min(jnp.where(sub16 == pos, bidx, _BIG),
                         axis=0, keepdims=True)
            vals.append(m)
            idxs.append(jnp.where(pos < _K, gi, c0 + pos - _K))
            if r < _K - 1:
                cv = jnp.where(subW == pos, _INF, cv)
        return (jnp.concatenate(vals, axis=0), jnp.concatenate(idxs, axis=0))

    init = (jnp.full((_K, _TR), _INF, jnp.float32),
            jnp.full((_K, _TR), _BIG, jnp.int32))
    _, bi = jax.lax.fori_loop(0, cnt_ref[i], col_body, init)

    selfm = bi == nodes
    p = jnp.min(jnp.where(selfm, sub16, _K), axis=0, keepdims=True)
    kept = jnp.where(sub16[:_K - 1, :] < p, bi[:_K - 1, :], bi[1:_K, :])
    out_src_ref[0:_K - 1, :] = kept
    out_src_ref[_K - 1:_K, :] = jnp.zeros((1, _TR), jnp.int32)
    deg = _K - jnp.sum(selfm.astype(jnp.int32), axis=0, keepdims=True)
    out_aux_ref[0:_K - 1, :] = jnp.broadcast_to(nodes, (_K - 1, _TR))
    out_aux_ref[_K - 1:_K, :] = deg


@jax.jit
def _knn_pallas(node_coord_src, batch_src):
    n = node_coord_src.shape[0]
    npad = ((n + _TC - 1) // _TC) * _TC
    nt = npad // _TR
    nb = 8

    coords8 = jnp.zeros((npad, 8), jnp.float32)
    coords8 = coords8.at[:n, :3].set(node_coord_src)
    coordsT = coords8.T
    brow = jnp.full((npad, 1), -1, jnp.int32).at[:n, 0].set(batch_src)
    bcol = jnp.full((1, npad), -2, jnp.int32).at[0, :n].set(batch_src)

    starts = jnp.searchsorted(batch_src, jnp.arange(nb, dtype=jnp.int32),
                              side="left").astype(jnp.int32)
    ends = jnp.searchsorted(batch_src, jnp.arange(nb, dtype=jnp.int32),
                            side="right").astype(jnp.int32)
    first = jnp.minimum(jnp.arange(nt, dtype=jnp.int32) * _TR, n - 1)
    last = jnp.minimum(first + _TR - 1, n - 1)
    lo_t = starts[batch_src[first]] // _TC
    hi_t = (ends[batch_src[last]] - 1) // _TC
    cnt_t = hi_t - lo_t + 1

    grid_spec = pltpu.PrefetchScalarGridSpec(
        num_scalar_prefetch=2,
        grid=(nt,),
        in_specs=[
            pl.BlockSpec((npad, 8), lambda i, lo, cnt: (0, 0)),
            pl.BlockSpec((8, npad), lambda i, lo, cnt: (0, 0)),
            pl.BlockSpec((npad, 1), lambda i, lo, cnt: (0, 0)),
            pl.BlockSpec((1, npad), lambda i, lo, cnt: (0, 0)),
        ],
        out_specs=[
            pl.BlockSpec((_K, _TR), lambda i, lo, cnt: (0, i)),
            pl.BlockSpec((_K, _TR), lambda i, lo, cnt: (0, i)),
        ],
    )
    out_src, out_aux = pl.pallas_call(
        _knn_body,
        grid_spec=grid_spec,
        out_shape=[
            jax.ShapeDtypeStruct((_K, nt * _TR), jnp.int32),
            jax.ShapeDtypeStruct((_K, nt * _TR), jnp.int32),
        ],
    )(lo_t, cnt_t, coords8, coordsT, brow, bcol)
    return out_src, out_aux


def kernel(node_coord_src, node_feature_src, batch_src):
    n = node_coord_src.shape[0]
    out_src, out_aux = _knn_pallas(node_coord_src, batch_src)
    edge_src = out_src[:_K - 1, :n].T.reshape(-1)
    edge_dst = out_aux[:_K - 1, :n].T.reshape(-1)
    degree = out_aux[_K - 1, :n]
    return (node_feature_src, node_coord_src, edge_src, edge_dst, degree,
            batch_src)
